# baseline (speedup 1.0000x reference)
import functools

import jax
import jax.numpy as jnp
from jax import lax
from jax.experimental import pallas as pl
from jax.experimental.pallas import tpu as pltpu
from jax.experimental.pallas import tpu_sc as plsc

B = 16384
D = 32
V = 1_000_000
NC = 2
NS = 16
NW = NC * NS
BPW = B // NW
CHUNKS = BPW // 16


def kernel(user_id, item_id, user_table, item_table):
    ut = user_table.T
    it = item_table.T

    mesh = plsc.VectorSubcoreMesh(core_axis_name="c", subcore_axis_name="s")

    @functools.partial(
        pl.kernel,
        mesh=mesh,
        out_type=jax.ShapeDtypeStruct((B,), jnp.float32),
        compiler_params=pltpu.CompilerParams(use_tc_tiling_on_sc=False),
        scratch_types=[
            pltpu.VMEM((BPW,), jnp.int32),
            pltpu.VMEM((BPW,), jnp.int32),
            pltpu.VMEM((D, BPW), jnp.float32),
            pltpu.VMEM((D, BPW), jnp.float32),
            pltpu.VMEM((BPW,), jnp.float32),
            pltpu.SemaphoreType.DMA,
        ],
    )
    def run(uid_hbm, iid_hbm, ut_hbm, it_hbm, out_hbm,
            uidx_v, iidx_v, ucols_v, icols_v, out_v, sem):
        wid = lax.axis_index("s") * NC + lax.axis_index("c")
        base = wid * BPW

        pltpu.sync_copy(uid_hbm.at[pl.ds(base, BPW)], uidx_v)
        pltpu.sync_copy(iid_hbm.at[pl.ds(base, BPW)], iidx_v)

        copies = []
        for d in range(D):
            copies.append(pltpu.async_copy(
                ut_hbm.at[d].at[uidx_v], ucols_v.at[d], sem))
            copies.append(pltpu.async_copy(
                it_hbm.at[d].at[iidx_v], icols_v.at[d], sem))
        for c in copies:
            c.wait()

        def chunk_body(c, carry):
            off = c * 16
            acc = jnp.zeros((16,), jnp.float32)
            for d in range(D):
                acc = acc + (ucols_v[d, pl.ds(off, 16)]
                             * icols_v[d, pl.ds(off, 16)])
            out_v[pl.ds(off, 16)] = acc
            return carry

        lax.fori_loop(0, CHUNKS, chunk_body, 0)

        pltpu.sync_copy(out_v, out_hbm.at[pl.ds(base, BPW)])

    return run(user_id, item_id, ut, it)


# speedup vs baseline: 19.0794x; 19.0794x over previous
import functools

import jax
import jax.numpy as jnp
from jax import lax
from jax.experimental import pallas as pl
from jax.experimental.pallas import tpu as pltpu
from jax.experimental.pallas import tpu_sc as plsc

B = 16384
D = 32
V = 1_000_000
NC = 2
NS = 16
NW = NC * NS
BPW = B // NW
CHUNKS = BPW // 16

JB = 4
W = 251_904
V2 = JB * W


def _detile(ut, it):
    def body(u_ref, i_ref, ou_ref, oi_ref):
        r = pl.program_id(1) % 8
        ou_ref[...] = u_ref[r, :]
        oi_ref[...] = i_ref[r, :]

    in_spec = pl.BlockSpec((8, W), lambda jb, d: (d // 8, jb))
    out_spec = pl.BlockSpec((W,), lambda jb, d: (d * JB + jb,))
    return pl.pallas_call(
        body,
        grid=(JB, D),
        in_specs=[in_spec, in_spec],
        out_specs=[out_spec, out_spec],
        out_shape=(jax.ShapeDtypeStruct((D * V2,), jnp.float32),
                   jax.ShapeDtypeStruct((D * V2,), jnp.float32)),
    )(ut, it)


def kernel(user_id, item_id, user_table, item_table):
    ut = user_table.T
    it = item_table.T
    u_lin, i_lin = _detile(ut, it)

    mesh = plsc.VectorSubcoreMesh(core_axis_name="c", subcore_axis_name="s")

    @functools.partial(
        pl.kernel,
        mesh=mesh,
        out_type=jax.ShapeDtypeStruct((B,), jnp.float32),
        compiler_params=pltpu.CompilerParams(use_tc_tiling_on_sc=False),
        scratch_types=[
            pltpu.VMEM((BPW,), jnp.int32),
            pltpu.VMEM((BPW,), jnp.int32),
            pltpu.VMEM((D, BPW), jnp.float32),
            pltpu.VMEM((D, BPW), jnp.float32),
            pltpu.VMEM((BPW,), jnp.float32),
            pltpu.SemaphoreType.DMA,
        ],
    )
    def run(uid_hbm, iid_hbm, ul_hbm, il_hbm, out_hbm,
            uidx_v, iidx_v, ucols_v, icols_v, out_v, sem):
        wid = lax.axis_index("s") * NC + lax.axis_index("c")
        base = wid * BPW

        pltpu.sync_copy(uid_hbm.at[pl.ds(base, BPW)], uidx_v)
        pltpu.sync_copy(iid_hbm.at[pl.ds(base, BPW)], iidx_v)

        copies = []
        for d in range(D):
            copies.append(pltpu.async_copy(
                ul_hbm.at[pl.ds(d * V2, V)].at[uidx_v], ucols_v.at[d], sem))
            copies.append(pltpu.async_copy(
                il_hbm.at[pl.ds(d * V2, V)].at[iidx_v], icols_v.at[d], sem))
        for c in copies:
            c.wait()

        def chunk_body(c, carry):
            off = c * 16
            acc = jnp.zeros((16,), jnp.float32)
            for d in range(D):
                acc = acc + (ucols_v[d, pl.ds(off, 16)]
                             * icols_v[d, pl.ds(off, 16)])
            out_v[pl.ds(off, 16)] = acc
            return carry

        lax.fori_loop(0, CHUNKS, chunk_body, 0)

        pltpu.sync_copy(out_v, out_hbm.at[pl.ds(base, BPW)])

    return run(user_id, item_id, u_lin, i_lin)


# speedup vs baseline: 22.9836x; 1.2046x over previous
import functools

import jax
import jax.numpy as jnp
from jax import lax
from jax.experimental import pallas as pl
from jax.experimental.pallas import tpu as pltpu
from jax.experimental.pallas import tpu_sc as plsc

B = 16384
D = 32
V = 1_000_000
NC = 2
NS = 16
NW = NC * NS
BPW = B // NW
CHUNKS = BPW // 16

W = 131_072
JB = 8
TPW = W // 128
BANDS = D // 8
BANDSZ = JB * W * 8
NTILES = BANDS * JB * TPW


def _detile(ut, it):
    def body(u_ref, i_ref, ou_ref, oi_ref):
        ou_ref[...] = jnp.swapaxes(u_ref[...].reshape(8, TPW, 128), 0, 1)
        oi_ref[...] = jnp.swapaxes(i_ref[...].reshape(8, TPW, 128), 0, 1)

    in_spec = pl.BlockSpec((8, W), lambda band, jb: (band, jb))
    out_spec = pl.BlockSpec((TPW, 8, 128),
                            lambda band, jb: (band * JB + jb, 0, 0))
    return pl.pallas_call(
        body,
        grid=(BANDS, JB),
        in_specs=[in_spec, in_spec],
        out_specs=[out_spec, out_spec],
        out_shape=(jax.ShapeDtypeStruct((NTILES, 8, 128), jnp.float32),
                   jax.ShapeDtypeStruct((NTILES, 8, 128), jnp.float32)),
    )(ut, it)


def kernel(user_id, item_id, user_table, item_table):
    ut = user_table.T
    it = item_table.T
    u3, i3 = _detile(ut, it)
    u_lin = u3.reshape(NTILES * 8 * 128)
    i_lin = i3.reshape(NTILES * 8 * 128)

    mesh = plsc.VectorSubcoreMesh(core_axis_name="c", subcore_axis_name="s")

    @functools.partial(
        pl.kernel,
        mesh=mesh,
        out_type=jax.ShapeDtypeStruct((B,), jnp.float32),
        compiler_params=pltpu.CompilerParams(use_tc_tiling_on_sc=False),
        scratch_types=[
            pltpu.VMEM((BPW,), jnp.int32),
            pltpu.VMEM((BPW,), jnp.int32),
            pltpu.VMEM((BPW,), jnp.int32),
            pltpu.VMEM((BPW,), jnp.int32),
            pltpu.VMEM((D, BPW), jnp.float32),
            pltpu.VMEM((D, BPW), jnp.float32),
            pltpu.VMEM((BPW,), jnp.float32),
            pltpu.SemaphoreType.DMA,
        ],
    )
    def run(uid_hbm, iid_hbm, ul_hbm, il_hbm, out_hbm,
            uidx_v, iidx_v, ug_v, ig_v, ucols_v, icols_v, out_v, sem):
        wid = lax.axis_index("s") * NC + lax.axis_index("c")
        base = wid * BPW

        pltpu.sync_copy(uid_hbm.at[pl.ds(base, BPW)], uidx_v)
        pltpu.sync_copy(iid_hbm.at[pl.ds(base, BPW)], iidx_v)

        def g_body(c, carry):
            off = c * 16
            for src, dst in ((uidx_v, ug_v), (iidx_v, ig_v)):
                j = src[pl.ds(off, 16)]
                g = ((j >> 17) << 20) | (((j >> 7) & 1023) << 10) | (j & 127)
                dst[pl.ds(off, 16)] = g
            return carry

        lax.fori_loop(0, CHUNKS, g_body, 0)

        copies = []
        for d in range(D):
            off0 = (d // 8) * BANDSZ + (d % 8) * 128
            copies.append(pltpu.async_copy(
                ul_hbm.at[pl.ds(off0, BANDSZ - (d % 8) * 128)].at[ug_v],
                ucols_v.at[d], sem))
            copies.append(pltpu.async_copy(
                il_hbm.at[pl.ds(off0, BANDSZ - (d % 8) * 128)].at[ig_v],
                icols_v.at[d], sem))
        for c in copies:
            c.wait()

        def chunk_body(c, carry):
            off = c * 16
            acc = jnp.zeros((16,), jnp.float32)
            for d in range(D):
                acc = acc + (ucols_v[d, pl.ds(off, 16)]
                             * icols_v[d, pl.ds(off, 16)])
            out_v[pl.ds(off, 16)] = acc
            return carry

        lax.fori_loop(0, CHUNKS, chunk_body, 0)

        pltpu.sync_copy(out_v, out_hbm.at[pl.ds(base, BPW)])

    return run(user_id, item_id, u_lin, i_lin)
